# Initial kernel scaffold; baseline (speedup 1.0000x reference)
import jax
import jax.numpy as jnp
from jax.experimental import pallas as pl


def kernel(x, edge_index, batch, edge_attr, W_node, b_node, W_edge, b_edge, W1, b1, gamma, beta, W2, b2, W_fc, b_fc):
    raise NotImplementedError("write your pallas kernel here")


# speedup vs baseline: 3.3738x; 3.3738x over previous
import functools

import jax
import jax.numpy as jnp
from jax import lax
from jax.experimental import pallas as pl
from jax.experimental.pallas import tpu as pltpu
from jax.experimental.pallas import tpu_sc as plsc

_N = 10000
_E = 320000
_D = 128
_DE = 16
_H = 64
_G = 64
_NL = 3

_NC = 2
_NS = 16
_NW = _NC * _NS
_EPW = _E // _NW
_C = 80
_NCHUNK = _EPW // _C
_NAGG = 10240
_RPS = _NAGG // _NS


def _encode_node(x, w, b):
    def body(x_ref, w_ref, b_ref, o_ref):
        o_ref[...] = jnp.dot(x_ref[...], w_ref[...],
                             preferred_element_type=jnp.float32, precision=lax.Precision.HIGHEST) + b_ref[...]
    return pl.pallas_call(
        body,
        out_shape=jax.ShapeDtypeStruct((_N, _H), jnp.float32),
    )(x, w, b)


def _encode_edge(edge_attr, w, b):
    blk = 4000
    def body(a_ref, w_ref, b_ref, o_ref):
        o_ref[...] = jnp.dot(a_ref[...], w_ref[...],
                             preferred_element_type=jnp.float32, precision=lax.Precision.HIGHEST) + b_ref[...]
    return pl.pallas_call(
        body,
        grid=(_E // blk,),
        in_specs=[pl.BlockSpec((blk, _DE), lambda i: (i, 0)),
                  pl.BlockSpec((_DE, _H), lambda i: (0, 0)),
                  pl.BlockSpec((1, _H), lambda i: (0, 0))],
        out_specs=pl.BlockSpec((blk, _H), lambda i: (i, 0)),
        out_shape=jax.ShapeDtypeStruct((_E, _H), jnp.float32),
    )(edge_attr, w, b)


def _mlp(h, agg, w1, b1, g, be, w2, b2):
    def body(h_ref, a_ref, w1_ref, b1_ref, g_ref, be_ref, w2_ref, b2_ref,
             o_ref):
        z = h_ref[...] + a_ref[0, :_N, :] + a_ref[1, :_N, :]
        z = jnp.dot(z, w1_ref[...], preferred_element_type=jnp.float32, precision=lax.Precision.HIGHEST)
        z = z + b1_ref[...]
        mean = jnp.mean(z, axis=0, keepdims=True)
        zc = z - mean
        var = jnp.mean(zc * zc, axis=0, keepdims=True)
        z = zc * (g_ref[...] * lax.rsqrt(var + 1e-5)) + be_ref[...]
        z = jnp.maximum(z, 0.0)
        z = jnp.dot(z, w2_ref[...], preferred_element_type=jnp.float32, precision=lax.Precision.HIGHEST)
        o_ref[...] = jnp.maximum(z + b2_ref[...], 0.0)
    return pl.pallas_call(
        body,
        out_shape=jax.ShapeDtypeStruct((_N, _H), jnp.float32),
    )(h, agg, w1, b1, g, be, w2, b2)


def _mlp_pool(h, agg, w1, b1, g, be, w2, b2, batch2d, wfc, bfc):
    def body(h_ref, a_ref, w1_ref, b1_ref, g_ref, be_ref, w2_ref, b2_ref,
             bat_ref, wfc_ref, bfc_ref, logits_ref, pooled_ref):
        z = h_ref[...] + a_ref[0, :_N, :] + a_ref[1, :_N, :]
        z = jnp.dot(z, w1_ref[...], preferred_element_type=jnp.float32, precision=lax.Precision.HIGHEST)
        z = z + b1_ref[...]
        mean = jnp.mean(z, axis=0, keepdims=True)
        zc = z - mean
        var = jnp.mean(zc * zc, axis=0, keepdims=True)
        z = zc * (g_ref[...] * lax.rsqrt(var + 1e-5)) + be_ref[...]
        z = jnp.maximum(z, 0.0)
        z = jnp.dot(z, w2_ref[...], preferred_element_type=jnp.float32, precision=lax.Precision.HIGHEST)
        hout = jnp.maximum(z + b2_ref[...], 0.0)
        gids = lax.broadcasted_iota(jnp.int32, (_G, _N), 0)
        onehot = (bat_ref[...] == gids).astype(jnp.float32)
        pooled = jnp.dot(onehot, hout, preferred_element_type=jnp.float32, precision=lax.Precision.HIGHEST)
        pooled_ref[...] = pooled
        logits_ref[...] = jnp.dot(pooled, wfc_ref[...],
                                  preferred_element_type=jnp.float32, precision=lax.Precision.HIGHEST) + bfc_ref[...]
    return pl.pallas_call(
        body,
        out_shape=(jax.ShapeDtypeStruct((_G, 1), jnp.float32),
                   jax.ShapeDtypeStruct((_G, _H), jnp.float32)),
    )(h, agg, w1, b1, g, be, w2, b2, batch2d, wfc, bfc)


def _msg_agg(h, ea, src, dst3):
    mesh = plsc.VectorSubcoreMesh(core_axis_name="c", subcore_axis_name="s",
                                  num_cores=_NC, num_subcores=_NS)

    @functools.partial(
        pl.kernel,
        out_type=jax.ShapeDtypeStruct((_NC, _NAGG, _H), jnp.float32),
        mesh=mesh,
        compiler_params=pltpu.CompilerParams(use_tc_tiling_on_sc=False),
        scratch_types=[
            pltpu.VMEM_SHARED((_NAGG, _H), jnp.float32),
            pltpu.VMEM((16, _H), jnp.float32),
            pltpu.VMEM((_EPW,), jnp.int32),
            pltpu.VMEM((_NCHUNK, _C), jnp.int32),
            pltpu.VMEM((_C, _H), jnp.float32),
            pltpu.VMEM((_C, _H), jnp.float32),
            pltpu.SemaphoreType.DMA,
        ],
    )
    def k(h_hbm, ea_hbm, src_hbm, dst_hbm, out_hbm,
          agg_sp, zero_v, src_v, dst_v, rows_v, ea_v, sem):
        cid = lax.axis_index("c")
        sid = lax.axis_index("s")
        wid = cid * _NS + sid

        zf = jnp.zeros((16,), jnp.float32)
        for r in range(16):
            for j in range(_H // 16):
                zero_v[r, pl.ds(j * 16, 16)] = zf
        def zbody(i, carry):
            pltpu.sync_copy(zero_v,
                            agg_sp.at[pl.ds(sid * _RPS + i * 16, 16)])
            return carry
        lax.fori_loop(0, _RPS // 16, zbody, 0)

        pltpu.sync_copy(src_hbm.at[pl.ds(wid * _EPW, _EPW)], src_v)
        pltpu.sync_copy(dst_hbm.at[wid], dst_v)
        plsc.subcore_barrier()

        def chunk(ci, carry):
            eoff = wid * _EPW + ci * _C
            pltpu.async_copy(h_hbm.at[src_v.at[pl.ds(ci * _C, _C)]],
                             rows_v, sem).wait()
            pltpu.sync_copy(ea_hbm.at[pl.ds(eoff, _C)], ea_v)
            def ebody(e, c2):
                for j in range(_H // 16):
                    sl = pl.ds(j * 16, 16)
                    rows_v[e, sl] = jnp.maximum(rows_v[e, sl] + ea_v[e, sl],
                                                0.0)
                return c2
            lax.fori_loop(0, _C, ebody, 0)
            pltpu.sync_copy(rows_v, agg_sp.at[dst_v.at[ci]], add=True)
            return carry
        lax.fori_loop(0, _NCHUNK, chunk, 0)
        plsc.subcore_barrier()

        pltpu.sync_copy(agg_sp.at[pl.ds(sid * _RPS, _RPS)],
                        out_hbm.at[cid, pl.ds(sid * _RPS, _RPS)])

    return k(h, ea, src, dst3)


def kernel(x, edge_index, batch, edge_attr, W_node, b_node, W_edge, b_edge,
           W1, b1, gamma, beta, W2, b2, W_fc, b_fc):
    src = edge_index[0]
    dst3 = edge_index[1].reshape(_NW, _NCHUNK, _C)
    b_node2 = b_node.reshape(1, _H)
    b_edge2 = b_edge.reshape(1, _H)
    batch2d = batch.reshape(1, _N)
    bfc2 = b_fc.reshape(1, 1)

    h = _encode_node(x, W_node, b_node2)
    ea = _encode_edge(edge_attr, W_edge, b_edge2)
    for i in range(_NL - 1):
        agg = _msg_agg(h, ea, src, dst3)
        h = _mlp(h, agg, W1[i], b1[i].reshape(1, _H), gamma[i].reshape(1, _H),
                 beta[i].reshape(1, _H), W2[i], b2[i].reshape(1, _H))
    agg = _msg_agg(h, ea, src, dst3)
    i = _NL - 1
    logits, pooled = _mlp_pool(
        h, agg, W1[i], b1[i].reshape(1, _H), gamma[i].reshape(1, _H),
        beta[i].reshape(1, _H), W2[i], b2[i].reshape(1, _H),
        batch2d, W_fc, bfc2)
    return (logits, pooled)


# speedup vs baseline: 5.7564x; 1.7062x over previous
import functools

import jax
import jax.numpy as jnp
from jax import lax
from jax.experimental import pallas as pl
from jax.experimental.pallas import tpu as pltpu
from jax.experimental.pallas import tpu_sc as plsc

_N = 10000
_E = 320000
_D = 128
_DE = 16
_H = 64
_G = 64
_NL = 3

_NC = 2
_NS = 16
_NW = _NC * _NS
_EPW = _E // _NW
_C = 80
_NCHUNK = _EPW // _C
_NAGG = 10240
_RPS = _NAGG // _NS
_NBUF = 5


def _encode_node(x, w, b):
    def body(x_ref, w_ref, b_ref, o_ref):
        o_ref[...] = jnp.dot(x_ref[...], w_ref[...],
                             preferred_element_type=jnp.float32, precision=lax.Precision.HIGHEST) + b_ref[...]
    return pl.pallas_call(
        body,
        out_shape=jax.ShapeDtypeStruct((_N, _H), jnp.float32),
    )(x, w, b)


def _encode_edge(edge_attr, w, b):
    blk = 4000
    def body(a_ref, w_ref, b_ref, o_ref):
        o_ref[...] = jnp.dot(a_ref[...], w_ref[...],
                             preferred_element_type=jnp.float32, precision=lax.Precision.HIGHEST) + b_ref[...]
    return pl.pallas_call(
        body,
        grid=(_E // blk,),
        in_specs=[pl.BlockSpec((blk, _DE), lambda i: (i, 0)),
                  pl.BlockSpec((_DE, _H), lambda i: (0, 0)),
                  pl.BlockSpec((1, _H), lambda i: (0, 0))],
        out_specs=pl.BlockSpec((blk, _H), lambda i: (i, 0)),
        out_shape=jax.ShapeDtypeStruct((_E, _H), jnp.float32),
    )(edge_attr, w, b)


def _mlp(h, agg, w1, b1, g, be, w2, b2):
    def body(h_ref, a_ref, w1_ref, b1_ref, g_ref, be_ref, w2_ref, b2_ref,
             o_ref):
        z = h_ref[...] + a_ref[0, :_N, :] + a_ref[1, :_N, :]
        z = jnp.dot(z, w1_ref[...], preferred_element_type=jnp.float32, precision=lax.Precision.HIGHEST)
        z = z + b1_ref[...]
        mean = jnp.mean(z, axis=0, keepdims=True)
        zc = z - mean
        var = jnp.mean(zc * zc, axis=0, keepdims=True)
        z = zc * (g_ref[...] * lax.rsqrt(var + 1e-5)) + be_ref[...]
        z = jnp.maximum(z, 0.0)
        z = jnp.dot(z, w2_ref[...], preferred_element_type=jnp.float32, precision=lax.Precision.HIGHEST)
        o_ref[...] = jnp.maximum(z + b2_ref[...], 0.0)
    return pl.pallas_call(
        body,
        out_shape=jax.ShapeDtypeStruct((_N, _H), jnp.float32),
    )(h, agg, w1, b1, g, be, w2, b2)


def _mlp_pool(h, agg, w1, b1, g, be, w2, b2, batch2d, wfc, bfc):
    def body(h_ref, a_ref, w1_ref, b1_ref, g_ref, be_ref, w2_ref, b2_ref,
             bat_ref, wfc_ref, bfc_ref, logits_ref, pooled_ref):
        z = h_ref[...] + a_ref[0, :_N, :] + a_ref[1, :_N, :]
        z = jnp.dot(z, w1_ref[...], preferred_element_type=jnp.float32, precision=lax.Precision.HIGHEST)
        z = z + b1_ref[...]
        mean = jnp.mean(z, axis=0, keepdims=True)
        zc = z - mean
        var = jnp.mean(zc * zc, axis=0, keepdims=True)
        z = zc * (g_ref[...] * lax.rsqrt(var + 1e-5)) + be_ref[...]
        z = jnp.maximum(z, 0.0)
        z = jnp.dot(z, w2_ref[...], preferred_element_type=jnp.float32, precision=lax.Precision.HIGHEST)
        hout = jnp.maximum(z + b2_ref[...], 0.0)
        gids = lax.broadcasted_iota(jnp.int32, (_G, _N), 0)
        onehot = (bat_ref[...] == gids).astype(jnp.float32)
        pooled = jnp.dot(onehot, hout, preferred_element_type=jnp.float32, precision=lax.Precision.HIGHEST)
        pooled_ref[...] = pooled
        logits_ref[...] = jnp.dot(pooled, wfc_ref[...],
                                  preferred_element_type=jnp.float32, precision=lax.Precision.HIGHEST) + bfc_ref[...]
    return pl.pallas_call(
        body,
        out_shape=(jax.ShapeDtypeStruct((_G, 1), jnp.float32),
                   jax.ShapeDtypeStruct((_G, _H), jnp.float32)),
    )(h, agg, w1, b1, g, be, w2, b2, batch2d, wfc, bfc)


def _msg_agg(h, ea, src, dst3):
    mesh = plsc.VectorSubcoreMesh(core_axis_name="c", subcore_axis_name="s",
                                  num_cores=_NC, num_subcores=_NS)

    @functools.partial(
        pl.kernel,
        out_type=jax.ShapeDtypeStruct((_NC, _NAGG, _H), jnp.float32),
        mesh=mesh,
        compiler_params=pltpu.CompilerParams(use_tc_tiling_on_sc=False),
        scratch_types=[
            pltpu.VMEM_SHARED((_NAGG, _H), jnp.float32),
            pltpu.VMEM((16, _H), jnp.float32),
            pltpu.VMEM((_EPW,), jnp.int32),
            pltpu.VMEM((_NCHUNK, _C), jnp.int32),
            pltpu.VMEM((_NBUF, _C, _H), jnp.float32),
            pltpu.VMEM((_NBUF, _C, _H), jnp.float32),
            pltpu.SemaphoreType.DMA((_NBUF,)),
        ],
    )
    def k(h_hbm, ea_hbm, src_hbm, dst_hbm, out_hbm,
          agg_sp, zero_v, src_v, dst_v, rows_v, ea_v, sem):
        cid = lax.axis_index("c")
        sid = lax.axis_index("s")
        wid = cid * _NS + sid

        zf = jnp.zeros((16,), jnp.float32)
        for r in range(16):
            for j in range(_H // 16):
                zero_v[r, pl.ds(j * 16, 16)] = zf
        def zbody(i, carry):
            pltpu.sync_copy(zero_v,
                            agg_sp.at[pl.ds(sid * _RPS + i * 16, 16)])
            return carry
        lax.fori_loop(0, _RPS // 16, zbody, 0)

        pltpu.sync_copy(src_hbm.at[pl.ds(wid * _EPW, _EPW)], src_v)
        pltpu.sync_copy(dst_hbm.at[wid], dst_v)
        plsc.subcore_barrier()

        def issue(b, ci):
            pltpu.async_copy(h_hbm.at[src_v.at[pl.ds(ci * _C, _C)]],
                             rows_v.at[b], sem.at[b])
            pltpu.async_copy(ea_hbm.at[pl.ds(wid * _EPW + ci * _C, _C)],
                             ea_v.at[b], sem.at[b])

        def wait(b, ci):
            pltpu.make_async_copy(h_hbm.at[src_v.at[pl.ds(ci * _C, _C)]],
                                  rows_v.at[b], sem.at[b]).wait()
            pltpu.make_async_copy(ea_hbm.at[pl.ds(wid * _EPW + ci * _C, _C)],
                                  ea_v.at[b], sem.at[b]).wait()

        for b in range(_NBUF):
            issue(b, jnp.int32(b))

        def group(i, carry):
            for b in range(_NBUF):
                ci = i * _NBUF + b
                wait(b, ci)
                rb = rows_v.at[b]
                eb = ea_v.at[b]

                def ebody(e, c2):
                    for j in range(_H // 16):
                        sl = pl.ds(j * 16, 16)
                        rb[e, sl] = jnp.maximum(rb[e, sl] + eb[e, sl], 0.0)
                    return c2
                lax.fori_loop(0, _C, ebody, 0)

                pltpu.sync_copy(rb, agg_sp.at[dst_v.at[ci]], add=True)
                issue(b, jnp.minimum(ci + _NBUF, _NCHUNK - 1))
            return carry
        lax.fori_loop(0, _NCHUNK // _NBUF, group, 0)
        for b in range(_NBUF):
            wait(b, jnp.int32(_NCHUNK - 1))
        plsc.subcore_barrier()

        pltpu.sync_copy(agg_sp.at[pl.ds(sid * _RPS, _RPS)],
                        out_hbm.at[cid, pl.ds(sid * _RPS, _RPS)])

    return k(h, ea, src, dst3)


def kernel(x, edge_index, batch, edge_attr, W_node, b_node, W_edge, b_edge,
           W1, b1, gamma, beta, W2, b2, W_fc, b_fc):
    src = edge_index[0]
    dst3 = edge_index[1].reshape(_NW, _NCHUNK, _C)
    b_node2 = b_node.reshape(1, _H)
    b_edge2 = b_edge.reshape(1, _H)
    batch2d = batch.reshape(1, _N)
    bfc2 = b_fc.reshape(1, 1)

    h = _encode_node(x, W_node, b_node2)
    ea = _encode_edge(edge_attr, W_edge, b_edge2)
    for i in range(_NL - 1):
        agg = _msg_agg(h, ea, src, dst3)
        h = _mlp(h, agg, W1[i], b1[i].reshape(1, _H), gamma[i].reshape(1, _H),
                 beta[i].reshape(1, _H), W2[i], b2[i].reshape(1, _H))
    agg = _msg_agg(h, ea, src, dst3)
    i = _NL - 1
    logits, pooled = _mlp_pool(
        h, agg, W1[i], b1[i].reshape(1, _H), gamma[i].reshape(1, _H),
        beta[i].reshape(1, _H), W2[i], b2[i].reshape(1, _H),
        batch2d, W_fc, bfc2)
    return (logits, pooled)


# speedup vs baseline: 7.6259x; 1.3248x over previous
import functools

import jax
import jax.numpy as jnp
from jax import lax
from jax.experimental import pallas as pl
from jax.experimental.pallas import tpu as pltpu
from jax.experimental.pallas import tpu_sc as plsc

_N = 10000
_E = 320000
_D = 128
_DE = 16
_H = 64
_G = 64
_NL = 3

_NC = 2
_NS = 16
_NW = _NC * _NS
_EPW = _E // _NW
_C = 80
_NCHUNK = _EPW // _C
_NAGG = 10240
_RPS = _NAGG // _NS
_NBUF = 5


def _encode_node(x, w, b):
    def body(x_ref, w_ref, b_ref, o_ref):
        o_ref[...] = jnp.dot(x_ref[...], w_ref[...],
                             preferred_element_type=jnp.float32, precision=lax.Precision.HIGHEST) + b_ref[...]
    return pl.pallas_call(
        body,
        out_shape=jax.ShapeDtypeStruct((_N, _H), jnp.float32),
    )(x, w, b)


def _encode_edge(attr8, wbd, bbd):
    blk = 1000
    def body(a_ref, w_ref, b_ref, o_ref):
        z = jnp.dot(a_ref[...], w_ref[...],
                    preferred_element_type=jnp.float32,
                    precision=lax.Precision.HIGHEST) + b_ref[...]
        o_ref[...] = z.reshape(blk * 4, 128)
    return pl.pallas_call(
        body,
        grid=(_E // 8 // blk,),
        in_specs=[pl.BlockSpec((blk, 128), lambda i: (i, 0)),
                  pl.BlockSpec((128, 512), lambda i: (0, 0)),
                  pl.BlockSpec((1, 512), lambda i: (0, 0))],
        out_specs=pl.BlockSpec((blk * 4, 128), lambda i: (i, 0)),
        out_shape=jax.ShapeDtypeStruct((_E // 2, 128), jnp.float32),
    )(attr8, wbd, bbd)


def _mlp(h, agg, w1, b1, g, be, w2, b2):
    def body(h_ref, a_ref, w1_ref, b1_ref, g_ref, be_ref, w2_ref, b2_ref,
             o_ref):
        z = h_ref[...] + a_ref[0, :_N, :] + a_ref[1, :_N, :]
        z = jnp.dot(z, w1_ref[...], preferred_element_type=jnp.float32, precision=lax.Precision.HIGHEST)
        z = z + b1_ref[...]
        mean = jnp.mean(z, axis=0, keepdims=True)
        zc = z - mean
        var = jnp.mean(zc * zc, axis=0, keepdims=True)
        z = zc * (g_ref[...] * lax.rsqrt(var + 1e-5)) + be_ref[...]
        z = jnp.maximum(z, 0.0)
        z = jnp.dot(z, w2_ref[...], preferred_element_type=jnp.float32, precision=lax.Precision.HIGHEST)
        o_ref[...] = jnp.maximum(z + b2_ref[...], 0.0)
    return pl.pallas_call(
        body,
        out_shape=jax.ShapeDtypeStruct((_N, _H), jnp.float32),
    )(h, agg, w1, b1, g, be, w2, b2)


def _mlp_pool(h, agg, w1, b1, g, be, w2, b2, batch2d, wfc, bfc):
    def body(h_ref, a_ref, w1_ref, b1_ref, g_ref, be_ref, w2_ref, b2_ref,
             bat_ref, wfc_ref, bfc_ref, logits_ref, pooled_ref):
        z = h_ref[...] + a_ref[0, :_N, :] + a_ref[1, :_N, :]
        z = jnp.dot(z, w1_ref[...], preferred_element_type=jnp.float32, precision=lax.Precision.HIGHEST)
        z = z + b1_ref[...]
        mean = jnp.mean(z, axis=0, keepdims=True)
        zc = z - mean
        var = jnp.mean(zc * zc, axis=0, keepdims=True)
        z = zc * (g_ref[...] * lax.rsqrt(var + 1e-5)) + be_ref[...]
        z = jnp.maximum(z, 0.0)
        z = jnp.dot(z, w2_ref[...], preferred_element_type=jnp.float32, precision=lax.Precision.HIGHEST)
        hout = jnp.maximum(z + b2_ref[...], 0.0)
        gids = lax.broadcasted_iota(jnp.int32, (_G, _N), 0)
        onehot = (bat_ref[...] == gids).astype(jnp.float32)
        pooled = jnp.dot(onehot, hout, preferred_element_type=jnp.float32, precision=lax.Precision.HIGHEST)
        pooled_ref[...] = pooled
        logits_ref[...] = jnp.dot(pooled, wfc_ref[...],
                                  preferred_element_type=jnp.float32, precision=lax.Precision.HIGHEST) + bfc_ref[...]
    return pl.pallas_call(
        body,
        out_shape=(jax.ShapeDtypeStruct((_G, 1), jnp.float32),
                   jax.ShapeDtypeStruct((_G, _H), jnp.float32)),
    )(h, agg, w1, b1, g, be, w2, b2, batch2d, wfc, bfc)


def _msg_agg(h, ea, src, dst3):
    mesh = plsc.VectorSubcoreMesh(core_axis_name="c", subcore_axis_name="s",
                                  num_cores=_NC, num_subcores=_NS)

    @functools.partial(
        pl.kernel,
        out_type=jax.ShapeDtypeStruct((_NC, _NAGG, _H), jnp.float32),
        mesh=mesh,
        compiler_params=pltpu.CompilerParams(use_tc_tiling_on_sc=False),
        scratch_types=[
            pltpu.VMEM_SHARED((_NAGG, _H), jnp.float32),
            pltpu.VMEM((16, _H), jnp.float32),
            pltpu.VMEM((_EPW,), jnp.int32),
            pltpu.VMEM((_NCHUNK, _C), jnp.int32),
            pltpu.VMEM((_NBUF, _C, _H), jnp.float32),
            pltpu.VMEM((_NBUF, _C // 2, 128), jnp.float32),
            pltpu.SemaphoreType.DMA((_NBUF,)),
        ],
    )
    def k(h_hbm, ea_hbm, src_hbm, dst_hbm, out_hbm,
          agg_sp, zero_v, src_v, dst_v, rows_v, ea_v, sem):
        cid = lax.axis_index("c")
        sid = lax.axis_index("s")
        wid = cid * _NS + sid

        zf = jnp.zeros((16,), jnp.float32)
        for r in range(16):
            for j in range(_H // 16):
                zero_v[r, pl.ds(j * 16, 16)] = zf
        def zbody(i, carry):
            pltpu.sync_copy(zero_v,
                            agg_sp.at[pl.ds(sid * _RPS + i * 16, 16)])
            return carry
        lax.fori_loop(0, _RPS // 16, zbody, 0)

        pltpu.sync_copy(src_hbm.at[pl.ds(wid * _EPW, _EPW)], src_v)
        pltpu.sync_copy(dst_hbm.at[wid], dst_v)
        plsc.subcore_barrier()

        def issue(b, ci):
            pltpu.async_copy(h_hbm.at[src_v.at[pl.ds(ci * _C, _C)]],
                             rows_v.at[b], sem.at[b])
            pltpu.async_copy(
                ea_hbm.at[pl.ds(wid * (_EPW // 2) + ci * (_C // 2), _C // 2)],
                ea_v.at[b], sem.at[b])

        def wait(b, ci):
            pltpu.make_async_copy(h_hbm.at[src_v.at[pl.ds(ci * _C, _C)]],
                                  rows_v.at[b], sem.at[b]).wait()
            pltpu.make_async_copy(
                ea_hbm.at[pl.ds(wid * (_EPW // 2) + ci * (_C // 2), _C // 2)],
                ea_v.at[b], sem.at[b]).wait()

        for b in range(_NBUF):
            issue(b, jnp.int32(b))

        def group(i, carry):
            for b in range(_NBUF):
                ci = i * _NBUF + b
                wait(b, ci)
                rb = rows_v.at[b]
                eb = ea_v.at[b]

                def ebody(dr, c2):
                    e0 = dr * 2
                    e1 = e0 + 1
                    for j in range(_H // 16):
                        sl = pl.ds(j * 16, 16)
                        rb[e0, sl] = jnp.maximum(
                            rb[e0, sl] + eb[dr, pl.ds(j * 16, 16)], 0.0)
                        rb[e1, sl] = jnp.maximum(
                            rb[e1, sl] + eb[dr, pl.ds(_H + j * 16, 16)], 0.0)
                    return c2
                lax.fori_loop(0, _C // 2, ebody, 0)

                pltpu.sync_copy(rb, agg_sp.at[dst_v.at[ci]], add=True)
                issue(b, jnp.minimum(ci + _NBUF, _NCHUNK - 1))
            return carry
        lax.fori_loop(0, _NCHUNK // _NBUF, group, 0)
        for b in range(_NBUF):
            wait(b, jnp.int32(_NCHUNK - 1))
        plsc.subcore_barrier()

        pltpu.sync_copy(agg_sp.at[pl.ds(sid * _RPS, _RPS)],
                        out_hbm.at[cid, pl.ds(sid * _RPS, _RPS)])

    return k(h, ea, src, dst3)


def kernel(x, edge_index, batch, edge_attr, W_node, b_node, W_edge, b_edge,
           W1, b1, gamma, beta, W2, b2, W_fc, b_fc):
    src = edge_index[0]
    dst3 = edge_index[1].reshape(_NW, _NCHUNK, _C)
    b_node2 = b_node.reshape(1, _H)
    batch2d = batch.reshape(1, _N)
    bfc2 = b_fc.reshape(1, 1)

    attr8 = edge_attr.reshape(_E // 8, 8 * _DE)
    wbd = jnp.zeros((8 * _DE, 8 * _H), W_edge.dtype)
    for t in range(8):
        wbd = wbd.at[t * _DE:(t + 1) * _DE, t * _H:(t + 1) * _H].set(W_edge)
    bbd = jnp.tile(b_edge, (8,)).reshape(1, 8 * _H)

    h = _encode_node(x, W_node, b_node2)
    ea = _encode_edge(attr8, wbd, bbd)
    for i in range(_NL - 1):
        agg = _msg_agg(h, ea, src, dst3)
        h = _mlp(h, agg, W1[i], b1[i].reshape(1, _H), gamma[i].reshape(1, _H),
                 beta[i].reshape(1, _H), W2[i], b2[i].reshape(1, _H))
    agg = _msg_agg(h, ea, src, dst3)
    i = _NL - 1
    logits, pooled = _mlp_pool(
        h, agg, W1[i], b1[i].reshape(1, _H), gamma[i].reshape(1, _H),
        beta[i].reshape(1, _H), W2[i], b2[i].reshape(1, _H),
        batch2d, W_fc, bfc2)
    return (logits, pooled)


# speedup vs baseline: 8.4754x; 1.1114x over previous
import functools

import jax
import jax.numpy as jnp
from jax import lax
from jax.experimental import pallas as pl
from jax.experimental.pallas import tpu as pltpu
from jax.experimental.pallas import tpu_sc as plsc

_N = 10000
_E = 320000
_D = 128
_DE = 16
_H = 64
_G = 64
_NL = 3

_NC = 2
_NS = 16
_NW = _NC * _NS
_EPW = _E // _NW
_C = 80
_NCHUNK = _EPW // _C
_NAGG = 10240
_RPS = _NAGG // _NS
_NBUF = 5


def _dot_ref(a, b):
    return jnp.dot(a.astype(jnp.bfloat16), b.astype(jnp.bfloat16),
                   preferred_element_type=jnp.float32)

def _encode_node(x, w, b):
    def body(x_ref, w_ref, b_ref, o_ref):
        o_ref[...] = _dot_ref(x_ref[...], w_ref[...]) + b_ref[...]
    return pl.pallas_call(
        body,
        out_shape=jax.ShapeDtypeStruct((_N, _H), jnp.float32),
    )(x, w, b)


def _encode_edge(attr8, wbd, bbd):
    blk = 1000
    def body(a_ref, w_ref, b_ref, o_ref):
        z = _dot_ref(a_ref[...], w_ref[...]) + b_ref[...]
        o_ref[...] = z.reshape(blk * 4, 128)
    return pl.pallas_call(
        body,
        grid=(_E // 8 // blk,),
        in_specs=[pl.BlockSpec((blk, 128), lambda i: (i, 0)),
                  pl.BlockSpec((128, 512), lambda i: (0, 0)),
                  pl.BlockSpec((1, 512), lambda i: (0, 0))],
        out_specs=pl.BlockSpec((blk * 4, 128), lambda i: (i, 0)),
        out_shape=jax.ShapeDtypeStruct((_E // 2, 128), jnp.float32),
    )(attr8, wbd, bbd)


def _mlp(h, agg, w1, b1, g, be, w2, b2):
    def body(h_ref, a_ref, w1_ref, b1_ref, g_ref, be_ref, w2_ref, b2_ref,
             o_ref):
        z = h_ref[...] + a_ref[0, :_N, :] + a_ref[1, :_N, :]
        z = _dot_ref(z, w1_ref[...])
        z = z + b1_ref[...]
        mean = jnp.mean(z, axis=0, keepdims=True)
        zc = z - mean
        var = jnp.mean(zc * zc, axis=0, keepdims=True)
        z = zc * (g_ref[...] * lax.rsqrt(var + 1e-5)) + be_ref[...]
        z = jnp.maximum(z, 0.0)
        z = _dot_ref(z, w2_ref[...])
        o_ref[...] = jnp.maximum(z + b2_ref[...], 0.0)
    return pl.pallas_call(
        body,
        out_shape=jax.ShapeDtypeStruct((_N, _H), jnp.float32),
    )(h, agg, w1, b1, g, be, w2, b2)


def _mlp_pool(h, agg, w1, b1, g, be, w2, b2, batch2d, wfc, bfc):
    def body(h_ref, a_ref, w1_ref, b1_ref, g_ref, be_ref, w2_ref, b2_ref,
             bat_ref, wfc_ref, bfc_ref, logits_ref, pooled_ref):
        z = h_ref[...] + a_ref[0, :_N, :] + a_ref[1, :_N, :]
        z = _dot_ref(z, w1_ref[...])
        z = z + b1_ref[...]
        mean = jnp.mean(z, axis=0, keepdims=True)
        zc = z - mean
        var = jnp.mean(zc * zc, axis=0, keepdims=True)
        z = zc * (g_ref[...] * lax.rsqrt(var + 1e-5)) + be_ref[...]
        z = jnp.maximum(z, 0.0)
        z = _dot_ref(z, w2_ref[...])
        hout = jnp.maximum(z + b2_ref[...], 0.0)
        gids = lax.broadcasted_iota(jnp.int32, (_G, _N), 0)
        onehot = (bat_ref[...] == gids).astype(jnp.float32)
        pooled = jnp.dot(onehot, hout, preferred_element_type=jnp.float32, precision=lax.Precision.HIGHEST)
        pooled_ref[...] = pooled
        logits_ref[...] = _dot_ref(pooled, wfc_ref[...]) + bfc_ref[...]
    return pl.pallas_call(
        body,
        out_shape=(jax.ShapeDtypeStruct((_G, 1), jnp.float32),
                   jax.ShapeDtypeStruct((_G, _H), jnp.float32)),
    )(h, agg, w1, b1, g, be, w2, b2, batch2d, wfc, bfc)


def _msg_agg(h, ea, src, dst3):
    mesh = plsc.VectorSubcoreMesh(core_axis_name="c", subcore_axis_name="s",
                                  num_cores=_NC, num_subcores=_NS)

    @functools.partial(
        pl.kernel,
        out_type=jax.ShapeDtypeStruct((_NC, _NAGG, _H), jnp.float32),
        mesh=mesh,
        compiler_params=pltpu.CompilerParams(use_tc_tiling_on_sc=False),
        scratch_types=[
            pltpu.VMEM_SHARED((_NAGG, _H), jnp.float32),
            pltpu.VMEM((16, _H), jnp.float32),
            pltpu.VMEM((_EPW,), jnp.int32),
            pltpu.VMEM((_NCHUNK, _C), jnp.int32),
            pltpu.VMEM((_NBUF, _C, _H), jnp.float32),
            pltpu.VMEM((_NBUF, _C // 2, 128), jnp.float32),
            pltpu.SemaphoreType.DMA((_NBUF,)),
        ],
    )
    def k(h_hbm, ea_hbm, src_hbm, dst_hbm, out_hbm,
          agg_sp, zero_v, src_v, dst_v, rows_v, ea_v, sem):
        cid = lax.axis_index("c")
        sid = lax.axis_index("s")
        wid = cid * _NS + sid

        zf = jnp.zeros((16,), jnp.float32)
        for r in range(16):
            for j in range(_H // 16):
                zero_v[r, pl.ds(j * 16, 16)] = zf
        def zbody(i, carry):
            pltpu.sync_copy(zero_v,
                            agg_sp.at[pl.ds(sid * _RPS + i * 16, 16)])
            return carry
        lax.fori_loop(0, _RPS // 16, zbody, 0)

        pltpu.sync_copy(src_hbm.at[pl.ds(wid * _EPW, _EPW)], src_v)
        pltpu.sync_copy(dst_hbm.at[wid], dst_v)
        plsc.subcore_barrier()

        def issue(b, ci):
            pltpu.async_copy(h_hbm.at[src_v.at[pl.ds(ci * _C, _C)]],
                             rows_v.at[b], sem.at[b])
            pltpu.async_copy(
                ea_hbm.at[pl.ds(wid * (_EPW // 2) + ci * (_C // 2), _C // 2)],
                ea_v.at[b], sem.at[b])

        def wait(b, ci):
            pltpu.make_async_copy(h_hbm.at[src_v.at[pl.ds(ci * _C, _C)]],
                                  rows_v.at[b], sem.at[b]).wait()
            pltpu.make_async_copy(
                ea_hbm.at[pl.ds(wid * (_EPW // 2) + ci * (_C // 2), _C // 2)],
                ea_v.at[b], sem.at[b]).wait()

        for b in range(_NBUF):
            issue(b, jnp.int32(b))

        def group(i, carry):
            for b in range(_NBUF):
                ci = i * _NBUF + b
                wait(b, ci)
                rb = rows_v.at[b]
                eb = ea_v.at[b]

                def ebody(dr, c2):
                    e0 = dr * 2
                    e1 = e0 + 1
                    for j in range(_H // 16):
                        sl = pl.ds(j * 16, 16)
                        rb[e0, sl] = jnp.maximum(
                            rb[e0, sl] + eb[dr, pl.ds(j * 16, 16)], 0.0)
                        rb[e1, sl] = jnp.maximum(
                            rb[e1, sl] + eb[dr, pl.ds(_H + j * 16, 16)], 0.0)
                    return c2
                lax.fori_loop(0, _C // 2, ebody, 0)

                pltpu.sync_copy(rb, agg_sp.at[dst_v.at[ci]], add=True)
                issue(b, jnp.minimum(ci + _NBUF, _NCHUNK - 1))
            return carry
        lax.fori_loop(0, _NCHUNK // _NBUF, group, 0)
        for b in range(_NBUF):
            wait(b, jnp.int32(_NCHUNK - 1))
        plsc.subcore_barrier()

        pltpu.sync_copy(agg_sp.at[pl.ds(sid * _RPS, _RPS)],
                        out_hbm.at[cid, pl.ds(sid * _RPS, _RPS)])

    return k(h, ea, src, dst3)


def kernel(x, edge_index, batch, edge_attr, W_node, b_node, W_edge, b_edge,
           W1, b1, gamma, beta, W2, b2, W_fc, b_fc):
    src = edge_index[0]
    dst3 = edge_index[1].reshape(_NW, _NCHUNK, _C)
    b_node2 = b_node.reshape(1, _H)
    batch2d = batch.reshape(1, _N)
    bfc2 = b_fc.reshape(1, 1)

    attr8 = edge_attr.reshape(_E // 8, 8 * _DE)
    wbd = jnp.zeros((8 * _DE, 8 * _H), W_edge.dtype)
    for t in range(8):
        wbd = wbd.at[t * _DE:(t + 1) * _DE, t * _H:(t + 1) * _H].set(W_edge)
    bbd = jnp.tile(b_edge, (8,)).reshape(1, 8 * _H)

    h = _encode_node(x, W_node, b_node2)
    ea = _encode_edge(attr8, wbd, bbd)
    for i in range(_NL - 1):
        agg = _msg_agg(h, ea, src, dst3)
        h = _mlp(h, agg, W1[i], b1[i].reshape(1, _H), gamma[i].reshape(1, _H),
                 beta[i].reshape(1, _H), W2[i], b2[i].reshape(1, _H))
    agg = _msg_agg(h, ea, src, dst3)
    i = _NL - 1
    logits, pooled = _mlp_pool(
        h, agg, W1[i], b1[i].reshape(1, _H), gamma[i].reshape(1, _H),
        beta[i].reshape(1, _H), W2[i], b2[i].reshape(1, _H),
        batch2d, W_fc, bfc2)
    return (logits, pooled)


# speedup vs baseline: 8.6257x; 1.0177x over previous
import functools

import jax
import jax.numpy as jnp
from jax import lax
from jax.experimental import pallas as pl
from jax.experimental.pallas import tpu as pltpu
from jax.experimental.pallas import tpu_sc as plsc

_N = 10000
_E = 320000
_D = 128
_DE = 16
_H = 64
_G = 64
_NL = 3

_NC = 2
_NS = 16
_NW = _NC * _NS
_EPW = _E // _NW
_C = 80
_NCHUNK = _EPW // _C
_NAGG = 10240
_RPS = _NAGG // _NS
_NBUF = 5


def _dot_ref(a, b):
    return jnp.dot(a, b, preferred_element_type=jnp.float32)

def _encode_node(x, w, b):
    def body(x_ref, w_ref, b_ref, o_ref):
        o_ref[...] = _dot_ref(x_ref[...], w_ref[...]) + b_ref[...]
    return pl.pallas_call(
        body,
        out_shape=jax.ShapeDtypeStruct((_N, _H), jnp.float32),
    )(x, w, b)


def _encode_edge(attr8, wbd, bbd):
    blk = 1000
    def body(a_ref, w_ref, b_ref, o_ref):
        z = _dot_ref(a_ref[...], w_ref[...]) + b_ref[...]
        o_ref[...] = z.reshape(blk * 4, 128)
    return pl.pallas_call(
        body,
        grid=(_E // 8 // blk,),
        in_specs=[pl.BlockSpec((blk, 128), lambda i: (i, 0)),
                  pl.BlockSpec((128, 512), lambda i: (0, 0)),
                  pl.BlockSpec((1, 512), lambda i: (0, 0))],
        out_specs=pl.BlockSpec((blk * 4, 128), lambda i: (i, 0)),
        out_shape=jax.ShapeDtypeStruct((_E // 2, 128), jnp.float32),
    )(attr8, wbd, bbd)


def _mlp(h, agg, w1, b1, g, be, w2, b2):
    def body(h_ref, a_ref, w1_ref, b1_ref, g_ref, be_ref, w2_ref, b2_ref,
             o_ref):
        z = h_ref[...] + a_ref[0, :_N, :] + a_ref[1, :_N, :]
        z = _dot_ref(z, w1_ref[...])
        z = z + b1_ref[...]
        mean = jnp.mean(z, axis=0, keepdims=True)
        zc = z - mean
        var = jnp.mean(zc * zc, axis=0, keepdims=True)
        z = zc * (g_ref[...] * lax.rsqrt(var + 1e-5)) + be_ref[...]
        z = jnp.maximum(z, 0.0)
        z = _dot_ref(z, w2_ref[...])
        o_ref[...] = jnp.maximum(z + b2_ref[...], 0.0)
    return pl.pallas_call(
        body,
        out_shape=jax.ShapeDtypeStruct((_N, _H), jnp.float32),
    )(h, agg, w1, b1, g, be, w2, b2)


def _mlp_pool(h, agg, w1, b1, g, be, w2, b2, batch2d, wfc, bfc):
    def body(h_ref, a_ref, w1_ref, b1_ref, g_ref, be_ref, w2_ref, b2_ref,
             bat_ref, wfc_ref, bfc_ref, logits_ref, pooled_ref):
        z = h_ref[...] + a_ref[0, :_N, :] + a_ref[1, :_N, :]
        z = _dot_ref(z, w1_ref[...])
        z = z + b1_ref[...]
        mean = jnp.mean(z, axis=0, keepdims=True)
        zc = z - mean
        var = jnp.mean(zc * zc, axis=0, keepdims=True)
        z = zc * (g_ref[...] * lax.rsqrt(var + 1e-5)) + be_ref[...]
        z = jnp.maximum(z, 0.0)
        z = _dot_ref(z, w2_ref[...])
        hout = jnp.maximum(z + b2_ref[...], 0.0)
        gids = lax.broadcasted_iota(jnp.int32, (_G, _N), 0)
        onehot = (bat_ref[...] == gids).astype(jnp.float32)
        pooled = jnp.dot(onehot, hout, preferred_element_type=jnp.float32, precision=lax.Precision.HIGHEST)
        pooled_ref[...] = pooled
        logits_ref[...] = _dot_ref(pooled, wfc_ref[...]) + bfc_ref[...]
    return pl.pallas_call(
        body,
        out_shape=(jax.ShapeDtypeStruct((_G, 1), jnp.float32),
                   jax.ShapeDtypeStruct((_G, _H), jnp.float32)),
    )(h, agg, w1, b1, g, be, w2, b2, batch2d, wfc, bfc)


def _msg_agg(h, ea, src, dst3):
    mesh = plsc.VectorSubcoreMesh(core_axis_name="c", subcore_axis_name="s",
                                  num_cores=_NC, num_subcores=_NS)

    @functools.partial(
        pl.kernel,
        out_type=jax.ShapeDtypeStruct((_NC, _NAGG, _H), jnp.float32),
        mesh=mesh,
        compiler_params=pltpu.CompilerParams(use_tc_tiling_on_sc=False),
        scratch_types=[
            pltpu.VMEM_SHARED((_NAGG, _H), jnp.float32),
            pltpu.VMEM((16, _H), jnp.float32),
            pltpu.VMEM((_EPW,), jnp.int32),
            pltpu.VMEM((_NCHUNK, _C), jnp.int32),
            pltpu.VMEM((_NBUF, _C, _H), jnp.float32),
            pltpu.VMEM((_NBUF, _C // 2, 128), jnp.float32),
            pltpu.SemaphoreType.DMA((_NBUF,)),
        ],
    )
    def k(h_hbm, ea_hbm, src_hbm, dst_hbm, out_hbm,
          agg_sp, zero_v, src_v, dst_v, rows_v, ea_v, sem):
        cid = lax.axis_index("c")
        sid = lax.axis_index("s")
        wid = cid * _NS + sid

        zf = jnp.zeros((16,), jnp.float32)
        for r in range(16):
            for j in range(_H // 16):
                zero_v[r, pl.ds(j * 16, 16)] = zf
        def zbody(i, carry):
            pltpu.sync_copy(zero_v,
                            agg_sp.at[pl.ds(sid * _RPS + i * 16, 16)])
            return carry
        lax.fori_loop(0, _RPS // 16, zbody, 0)

        pltpu.sync_copy(src_hbm.at[pl.ds(wid * _EPW, _EPW)], src_v)
        pltpu.sync_copy(dst_hbm.at[wid], dst_v)
        plsc.subcore_barrier()

        def issue(b, ci):
            pltpu.async_copy(h_hbm.at[src_v.at[pl.ds(ci * _C, _C)]],
                             rows_v.at[b], sem.at[b])
            pltpu.async_copy(
                ea_hbm.at[pl.ds(wid * (_EPW // 2) + ci * (_C // 2), _C // 2)],
                ea_v.at[b], sem.at[b])

        def wait(b, ci):
            pltpu.make_async_copy(h_hbm.at[src_v.at[pl.ds(ci * _C, _C)]],
                                  rows_v.at[b], sem.at[b]).wait()
            pltpu.make_async_copy(
                ea_hbm.at[pl.ds(wid * (_EPW // 2) + ci * (_C // 2), _C // 2)],
                ea_v.at[b], sem.at[b]).wait()

        for b in range(_NBUF):
            issue(b, jnp.int32(b))

        def group(i, carry):
            for b in range(_NBUF):
                ci = i * _NBUF + b
                wait(b, ci)
                rb = rows_v.at[b]
                eb = ea_v.at[b]

                def ebody(dr, c2):
                    e0 = dr * 2
                    e1 = e0 + 1
                    for j in range(_H // 16):
                        sl = pl.ds(j * 16, 16)
                        rb[e0, sl] = jnp.maximum(
                            rb[e0, sl] + eb[dr, pl.ds(j * 16, 16)], 0.0)
                        rb[e1, sl] = jnp.maximum(
                            rb[e1, sl] + eb[dr, pl.ds(_H + j * 16, 16)], 0.0)
                    return c2
                lax.fori_loop(0, _C // 2, ebody, 0)

                pltpu.sync_copy(rb, agg_sp.at[dst_v.at[ci]], add=True)
                issue(b, jnp.minimum(ci + _NBUF, _NCHUNK - 1))
            return carry
        lax.fori_loop(0, _NCHUNK // _NBUF, group, 0)
        for b in range(_NBUF):
            wait(b, jnp.int32(_NCHUNK - 1))
        plsc.subcore_barrier()

        pltpu.sync_copy(agg_sp.at[pl.ds(sid * _RPS, _RPS)],
                        out_hbm.at[cid, pl.ds(sid * _RPS, _RPS)])

    return k(h, ea, src, dst3)


def kernel(x, edge_index, batch, edge_attr, W_node, b_node, W_edge, b_edge,
           W1, b1, gamma, beta, W2, b2, W_fc, b_fc):
    src = edge_index[0]
    dst3 = edge_index[1].reshape(_NW, _NCHUNK, _C)
    b_node2 = b_node.reshape(1, _H)
    batch2d = batch.reshape(1, _N)
    bfc2 = b_fc.reshape(1, 1)

    attr8 = edge_attr.reshape(_E // 8, 8 * _DE)
    wbd = jnp.zeros((8 * _DE, 8 * _H), W_edge.dtype)
    for t in range(8):
        wbd = wbd.at[t * _DE:(t + 1) * _DE, t * _H:(t + 1) * _H].set(W_edge)
    bbd = jnp.tile(b_edge, (8,)).reshape(1, 8 * _H)

    h = _encode_node(x, W_node, b_node2)
    ea = _encode_edge(attr8, wbd, bbd)
    for i in range(_NL - 1):
        agg = _msg_agg(h, ea, src, dst3)
        h = _mlp(h, agg, W1[i], b1[i].reshape(1, _H), gamma[i].reshape(1, _H),
                 beta[i].reshape(1, _H), W2[i], b2[i].reshape(1, _H))
    agg = _msg_agg(h, ea, src, dst3)
    i = _NL - 1
    logits, pooled = _mlp_pool(
        h, agg, W1[i], b1[i].reshape(1, _H), gamma[i].reshape(1, _H),
        beta[i].reshape(1, _H), W2[i], b2[i].reshape(1, _H),
        batch2d, W_fc, bfc2)
    return (logits, pooled)
